# baseline (speedup 1.0000x reference)
import functools

import jax
import jax.numpy as jnp
from jax import lax
from jax.experimental import pallas as pl
from jax.experimental.pallas import tpu as pltpu
from jax.experimental.pallas import tpu_sc as plsc

B, L, D = 4096, 200, 64
NC, NS = 2, 16
NW = NC * NS
ROWS_W = B // NW
SPLITS = ((0, 128), (128, 72))
R = 2
NT = ROWS_W // R
GROUP = R * L

_mesh = plsc.VectorSubcoreMesh(core_axis_name="c", subcore_axis_name="s")


@functools.partial(
    pl.kernel,
    out_type=jax.ShapeDtypeStruct((B, L, D), jnp.float32),
    mesh=_mesh,
    compiler_params=pltpu.CompilerParams(use_tc_tiling_on_sc=False),
    scratch_types=[
        pltpu.VMEM((ROWS_W, L), jnp.int32),
        pltpu.VMEM((2, R, L, D), jnp.float32),
        pltpu.SemaphoreType.DMA,
        pltpu.SemaphoreType.DMA,
    ],
)
def _gather(idx_hbm, table_hbm, out_hbm, idx_v, rows_v, gsem, ssem):
  wid = lax.axis_index("s") * NC + lax.axis_index("c")
  row0 = wid * ROWS_W
  pltpu.sync_copy(idx_hbm.at[pl.ds(row0, ROWS_W)], idx_v)

  def start_group(t, p):
    for r in range(R):
      for off, size in SPLITS:
        pltpu.async_copy(
            table_hbm.at[idx_v.at[t * R + r, pl.ds(off, size)]],
            rows_v.at[p, r, pl.ds(off, size)],
            gsem,
        )

  def wait_gathers():
    pltpu.make_async_copy(out_hbm.at[pl.ds(0, R)], rows_v.at[0], gsem).wait()

  def wait_store():
    pltpu.make_async_copy(rows_v.at[0], out_hbm.at[pl.ds(0, R)], ssem).wait()

  start_group(0, 0)

  def body(t, carry):
    p = lax.rem(t, 2)
    wait_gathers()

    @pl.when(t + 1 < NT)
    def _():
      @pl.when(t >= 1)
      def _():
        wait_store()
      start_group(t + 1, 1 - p)

    pltpu.async_copy(
        rows_v.at[p],
        out_hbm.at[pl.ds(row0 + t * R, R)],
        ssem,
    )
    return carry

  lax.fori_loop(0, NT, body, 0)
  wait_store()
  wait_store()


def kernel(indices, table):
  return _gather(indices.astype(jnp.int32), table)


# speedup vs baseline: 1.2108x; 1.2108x over previous
import functools

import jax
import jax.numpy as jnp
from jax import lax
from jax.experimental import pallas as pl
from jax.experimental.pallas import tpu as pltpu
from jax.experimental.pallas import tpu_sc as plsc

B, L, D = 4096, 200, 64
DP = 128
NC, NS = 2, 16
NW = NC * NS
ROWS_W = B // NW
SPLITS = ((0, 128), (128, 72))
R = 1
NT = ROWS_W // R

_mesh = plsc.VectorSubcoreMesh(core_axis_name="c", subcore_axis_name="s")


@functools.partial(
    pl.kernel,
    out_type=jax.ShapeDtypeStruct((B, L, DP), jnp.float32),
    mesh=_mesh,
    compiler_params=pltpu.CompilerParams(use_tc_tiling_on_sc=True),
    scratch_types=[
        pltpu.VMEM((ROWS_W, L), jnp.int32),
        pltpu.VMEM((2, R, L, DP), jnp.float32),
        pltpu.SemaphoreType.DMA,
        pltpu.SemaphoreType.DMA,
    ],
)
def _gather(idx_hbm, table_hbm, out_hbm, idx_v, rows_v, gsem, ssem):
  wid = lax.axis_index("s") * NC + lax.axis_index("c")
  row0 = wid * ROWS_W
  pltpu.sync_copy(idx_hbm.at[pl.ds(row0, ROWS_W)], idx_v)

  def start_group(t, p):
    for r in range(R):
      for off, size in SPLITS:
        pltpu.async_copy(
            table_hbm.at[idx_v.at[t * R + r, pl.ds(off, size)]],
            rows_v.at[p, r, pl.ds(off, size)],
            gsem,
        )

  def wait_gathers():
    pltpu.make_async_copy(
        table_hbm.at[pl.ds(0, R * L)], rows_v.at[0], gsem
    ).wait()

  def wait_store():
    pltpu.make_async_copy(rows_v.at[0], out_hbm.at[pl.ds(0, R)], ssem).wait()

  start_group(0, 0)

  def body(t, carry):
    p = lax.rem(t, 2)
    wait_gathers()

    @pl.when(t + 1 < NT)
    def _():
      @pl.when(t >= 1)
      def _():
        wait_store()
      start_group(t + 1, 1 - p)

    pltpu.async_copy(
        rows_v.at[p],
        out_hbm.at[pl.ds(row0 + t * R, R)],
        ssem,
    )
    return carry

  lax.fori_loop(0, NT, body, 0)
  wait_store()
  wait_store()


def kernel(indices, table):
  table_pad = jnp.pad(table, ((0, 0), (0, DP - D)))
  out = _gather(indices.astype(jnp.int32), table_pad)
  return out[:, :, :D]


# speedup vs baseline: 1.2293x; 1.0153x over previous
import functools

import jax
import jax.numpy as jnp
from jax import lax
from jax.experimental import pallas as pl
from jax.experimental.pallas import tpu as pltpu
from jax.experimental.pallas import tpu_sc as plsc

B, L, D = 4096, 200, 64
DP = 128
NC, NS = 2, 16
NW = NC * NS
ROWS_W = B // NW
SPLITS = ((0, 128), (128, 72))
R = 1
NT = ROWS_W // R
NBUF = 3

_mesh = plsc.VectorSubcoreMesh(core_axis_name="c", subcore_axis_name="s")


@functools.partial(
    pl.kernel,
    out_type=jax.ShapeDtypeStruct((B, L, DP), jnp.float32),
    mesh=_mesh,
    compiler_params=pltpu.CompilerParams(use_tc_tiling_on_sc=True),
    scratch_types=[
        pltpu.VMEM((ROWS_W, L), jnp.int32),
        pltpu.VMEM((NBUF, R, L, DP), jnp.float32),
        [pltpu.SemaphoreType.DMA] * NBUF,
        [pltpu.SemaphoreType.DMA] * NBUF,
    ],
)
def _gather(idx_hbm, table_hbm, out_hbm, idx_v, rows_v, gsems, ssems):
  wid = lax.axis_index("s") * NC + lax.axis_index("c")
  row0 = wid * ROWS_W
  pltpu.sync_copy(idx_hbm.at[pl.ds(row0, ROWS_W)], idx_v)

  def start_group(t, b):
    for r in range(R):
      for off, size in SPLITS:
        pltpu.async_copy(
            table_hbm.at[idx_v.at[t * R + r, pl.ds(off, size)]],
            rows_v.at[b, r, pl.ds(off, size)],
            gsems[b],
        )

  def wait_gathers(b):
    pltpu.make_async_copy(
        table_hbm.at[pl.ds(0, R * L)], rows_v.at[0], gsems[b]
    ).wait()

  def wait_store(b):
    pltpu.make_async_copy(
        rows_v.at[0], out_hbm.at[pl.ds(0, R)], ssems[b]
    ).wait()

  def dispatch(t, fns):
    for b in range(NBUF):
      @pl.when(lax.rem(t, NBUF) == b)
      def _(b=b):
        fns(b)

  start_group(0, 0)
  start_group(1, 1)

  def body(t, carry):
    def slot(b):
      wait_gathers(b)

      @pl.when(t + 2 < NT)
      def _():
        b2 = (b + 2) % NBUF

        @pl.when(t >= 1)
        def _():
          wait_store(b2)
        start_group(t + 2, b2)

      pltpu.async_copy(
          rows_v.at[b],
          out_hbm.at[pl.ds(row0 + t * R, R)],
          ssems[b],
      )

    dispatch(t, slot)
    return carry

  lax.fori_loop(0, NT, body, 0)
  wait_store((NT - 2) % NBUF)
  wait_store((NT - 1) % NBUF)


def kernel(indices, table):
  table_pad = jnp.pad(table, ((0, 0), (0, DP - D)))
  out = _gather(indices.astype(jnp.int32), table_pad)
  return out[:, :, :D]
